# baseline (speedup 1.0000x reference)
import functools

import jax
import jax.numpy as jnp
from jax import lax
from jax.experimental import pallas as pl
from jax.experimental.pallas import tpu as pltpu
from jax.experimental.pallas import tpu_sc as plsc

B, S = 4, 8192
TOK = B * S
IN_D = 768
PROJ_D = 512
EMB_D = 128
OUT_D = PROJ_D + 2 * EMB_D

NC, NS = 2, 16
NW = NC * NS
TPW = TOK // NW
CH = 128
NCH = TPW // CH

M_TILE = 512


def _sc_gather_body(tag_idx_hbm, pred_idx_hbm, tag_tab_hbm, pred_tab_hbm,
                    tag_out_hbm, pred_out_hbm,
                    idx_t, idx_p, rows_t0, rows_t1, rows_p0, rows_p1,
                    sem_t0, sem_t1, sem_p0, sem_p1):
    wid = lax.axis_index("s") * NC + lax.axis_index("c")
    base = wid * NCH
    pltpu.sync_copy(tag_idx_hbm.at[pl.ds(base, NCH)], idx_t)
    pltpu.sync_copy(pred_idx_hbm.at[pl.ds(base, NCH)], idx_p)
    rows_t = (rows_t0, rows_t1)
    rows_p = (rows_p0, rows_p1)
    sems_t = (sem_t0, sem_t1)
    sems_p = (sem_p0, sem_p1)
    cps = {}
    for j in range(NCH + 1):
        s = j % 2
        if j < NCH:
            cps[(j, "t")] = pltpu.async_copy(
                tag_tab_hbm.at[idx_t.at[j]], rows_t[s], sems_t[s])
            cps[(j, "p")] = pltpu.async_copy(
                pred_tab_hbm.at[idx_p.at[j]], rows_p[s], sems_p[s])
        if j > 0:
            k = j - 1
            ks = k % 2
            row0 = (base + k) * CH
            cps.pop((k, "t")).wait()
            pltpu.sync_copy(rows_t[ks], tag_out_hbm.at[pl.ds(row0, CH)])
            cps.pop((k, "p")).wait()
            pltpu.sync_copy(rows_p[ks], pred_out_hbm.at[pl.ds(row0, CH)])


@functools.partial(jax.jit, static_argnums=())
def _sc_gather(tag_idx, pred_idx, tag_tab, pred_tab):
    mesh = plsc.VectorSubcoreMesh(core_axis_name="c", subcore_axis_name="s")
    return pl.kernel(
        _sc_gather_body,
        out_type=(
            jax.ShapeDtypeStruct((TOK, EMB_D), jnp.float32),
            jax.ShapeDtypeStruct((TOK, EMB_D), jnp.float32),
        ),
        mesh=mesh,
        scratch_types=[
            pltpu.VMEM((NCH, CH), jnp.int32),
            pltpu.VMEM((NCH, CH), jnp.int32),
            pltpu.VMEM((CH, EMB_D), jnp.float32),
            pltpu.VMEM((CH, EMB_D), jnp.float32),
            pltpu.VMEM((CH, EMB_D), jnp.float32),
            pltpu.VMEM((CH, EMB_D), jnp.float32),
            pltpu.SemaphoreType.DMA,
            pltpu.SemaphoreType.DMA,
            pltpu.SemaphoreType.DMA,
            pltpu.SemaphoreType.DMA,
        ],
    )(tag_idx, pred_idx, tag_tab, pred_tab)


def _tc_body(x_ref, w_ref, b_ref, tag_ref, pred_ref, out_ref):
    acc = jnp.dot(x_ref[...], w_ref[...], preferred_element_type=jnp.float32)
    out_ref[:, :PROJ_D] = acc + b_ref[...]
    out_ref[:, PROJ_D:PROJ_D + EMB_D] = tag_ref[...]
    out_ref[:, PROJ_D + EMB_D:] = pred_ref[...]


def _tc_project_concat(x2d, W, b2d, tag_emb, pred_emb):
    return pl.pallas_call(
        _tc_body,
        grid=(TOK // M_TILE,),
        in_specs=[
            pl.BlockSpec((M_TILE, IN_D), lambda i: (i, 0)),
            pl.BlockSpec((IN_D, PROJ_D), lambda i: (0, 0)),
            pl.BlockSpec((1, PROJ_D), lambda i: (0, 0)),
            pl.BlockSpec((M_TILE, EMB_D), lambda i: (i, 0)),
            pl.BlockSpec((M_TILE, EMB_D), lambda i: (i, 0)),
        ],
        out_specs=pl.BlockSpec((M_TILE, OUT_D), lambda i: (i, 0)),
        out_shape=jax.ShapeDtypeStruct((TOK, OUT_D), jnp.float32),
    )(x2d, W, b2d, tag_emb, pred_emb)


def kernel(input_layer, tag_ids, predicate_mask, tag_embeddings,
           predicate_embeddings, W, b):
    x2d = input_layer.reshape(TOK, IN_D)
    tag_idx = tag_ids.astype(jnp.int32).reshape(NW * NCH, CH)
    pred_idx = predicate_mask.astype(jnp.int32).reshape(NW * NCH, CH)
    tag_emb, pred_emb = _sc_gather(tag_idx, pred_idx, tag_embeddings,
                                   predicate_embeddings)
    out = _tc_project_concat(x2d, W, b.reshape(1, PROJ_D), tag_emb, pred_emb)
    return out.reshape(B, S, OUT_D)


# speedup vs baseline: 6.0483x; 6.0483x over previous
import functools

import jax
import jax.numpy as jnp
from jax import lax
from jax.experimental import pallas as pl
from jax.experimental.pallas import tpu as pltpu
from jax.experimental.pallas import tpu_sc as plsc

B, S = 4, 8192
TOK = B * S
IN_D = 768
PROJ_D = 512
EMB_D = 128
OUT_D = PROJ_D + 2 * EMB_D

NC, NS = 2, 16
NW = NC * NS
TPW = TOK // NW
CH = 128
NCH = TPW // CH
NBUF = 4

M_TILE = 512


def _sc_gather_body(tag_idx_hbm, tag_tab_hbm, tag_out_hbm,
                    idx_t, r0, r1, r2, r3, s0, s1, s2, s3):
    wid = lax.axis_index("s") * NC + lax.axis_index("c")
    base = wid * NCH
    pltpu.sync_copy(tag_idx_hbm.at[pl.ds(base, NCH)], idx_t)
    rows = (r0, r1, r2, r3)
    sems = (s0, s1, s2, s3)

    def fire(j):
        return pltpu.async_copy(
            tag_tab_hbm.at[idx_t.at[j]], rows[j % NBUF], sems[j % NBUF])

    cps = {}
    for j in range(NBUF):
        cps[j] = fire(j)
    for j in range(NCH):
        cps.pop(j).wait()
        pltpu.sync_copy(rows[j % NBUF],
                        tag_out_hbm.at[pl.ds((base + j) * CH, CH)])
        if j + NBUF < NCH:
            cps[j + NBUF] = fire(j + NBUF)


def _sc_gather(tag_idx, tag_tab):
    mesh = plsc.VectorSubcoreMesh(core_axis_name="c", subcore_axis_name="s")
    return pl.kernel(
        _sc_gather_body,
        out_type=jax.ShapeDtypeStruct((TOK, EMB_D), jnp.float32),
        mesh=mesh,
        scratch_types=[
            pltpu.VMEM((NCH, CH), jnp.int32),
            pltpu.VMEM((CH, EMB_D), jnp.float32),
            pltpu.VMEM((CH, EMB_D), jnp.float32),
            pltpu.VMEM((CH, EMB_D), jnp.float32),
            pltpu.VMEM((CH, EMB_D), jnp.float32),
            pltpu.SemaphoreType.DMA,
            pltpu.SemaphoreType.DMA,
            pltpu.SemaphoreType.DMA,
            pltpu.SemaphoreType.DMA,
        ],
    )(tag_idx, tag_tab)


def _tc_body(x_ref, w_ref, b_ref, tag_ref, mask_ref, ptab_ref, out_ref):
    acc = jnp.dot(x_ref[...], w_ref[...], preferred_element_type=jnp.float32)
    out_ref[:, :PROJ_D] = acc + b_ref[...]
    out_ref[:, PROJ_D:PROJ_D + EMB_D] = tag_ref[...]
    pred = jnp.where(mask_ref[...] == 0, ptab_ref[0:1, :], ptab_ref[1:2, :])
    out_ref[:, PROJ_D + EMB_D:] = pred


def _tc_project_concat(x2d, W, b2d, tag_emb, mask_col, ptab):
    return pl.pallas_call(
        _tc_body,
        grid=(TOK // M_TILE,),
        in_specs=[
            pl.BlockSpec((M_TILE, IN_D), lambda i: (i, 0)),
            pl.BlockSpec((IN_D, PROJ_D), lambda i: (0, 0)),
            pl.BlockSpec((1, PROJ_D), lambda i: (0, 0)),
            pl.BlockSpec((M_TILE, EMB_D), lambda i: (i, 0)),
            pl.BlockSpec((M_TILE, 1), lambda i: (i, 0)),
            pl.BlockSpec((8, EMB_D), lambda i: (0, 0)),
        ],
        out_specs=pl.BlockSpec((M_TILE, OUT_D), lambda i: (i, 0)),
        out_shape=jax.ShapeDtypeStruct((TOK, OUT_D), jnp.float32),
    )(x2d, W, b2d, tag_emb, mask_col, ptab)


def kernel(input_layer, tag_ids, predicate_mask, tag_embeddings,
           predicate_embeddings, W, b):
    x2d = input_layer.reshape(TOK, IN_D)
    tag_idx = tag_ids.astype(jnp.int32).reshape(NW * NCH, CH)
    mask_col = predicate_mask.astype(jnp.int32).reshape(TOK, 1)
    ptab = jnp.zeros((8, EMB_D), jnp.float32).at[:2].set(predicate_embeddings)
    tag_emb = _sc_gather(tag_idx, tag_embeddings)
    out = _tc_project_concat(x2d, W, b.reshape(1, PROJ_D), tag_emb,
                             mask_col, ptab)
    return out.reshape(B, S, OUT_D)


# speedup vs baseline: 6.9931x; 1.1562x over previous
import functools

import jax
import jax.numpy as jnp
from jax import lax
from jax.experimental import pallas as pl
from jax.experimental.pallas import tpu as pltpu
from jax.experimental.pallas import tpu_sc as plsc

B, S = 4, 8192
TOK = B * S
IN_D = 768
PROJ_D = 512
EMB_D = 128
OUT_D = PROJ_D + 2 * EMB_D

NC, NS = 2, 16
NW = NC * NS
TPW = TOK // NW
CH = 128
NCH = TPW // CH
NBUF = 4

M_TILE = 1024


def _sc_gather_body(tag_idx_hbm, tag_tab_hbm, tag_out_hbm,
                    idx_t, r0, r1, r2, r3, s0, s1, s2, s3):
    wid = lax.axis_index("s") * NC + lax.axis_index("c")
    base = wid * NCH
    pltpu.sync_copy(tag_idx_hbm.at[pl.ds(base, NCH)], idx_t)
    rows = (r0, r1, r2, r3)
    sems = (s0, s1, s2, s3)

    def fire(j):
        return pltpu.async_copy(
            tag_tab_hbm.at[idx_t.at[j]], rows[j % NBUF], sems[j % NBUF])

    cps = {}
    for j in range(NBUF):
        cps[j] = fire(j)
    for j in range(NCH):
        cps.pop(j).wait()
        pltpu.sync_copy(rows[j % NBUF],
                        tag_out_hbm.at[pl.ds((base + j) * CH, CH)])
        if j + NBUF < NCH:
            cps[j + NBUF] = fire(j + NBUF)


def _sc_gather(tag_idx, tag_tab):
    mesh = plsc.VectorSubcoreMesh(core_axis_name="c", subcore_axis_name="s")
    return pl.kernel(
        _sc_gather_body,
        out_type=jax.ShapeDtypeStruct((TOK, EMB_D), jnp.float32),
        mesh=mesh,
        scratch_types=[
            pltpu.VMEM((NCH, CH), jnp.int32),
            pltpu.VMEM((CH, EMB_D), jnp.float32),
            pltpu.VMEM((CH, EMB_D), jnp.float32),
            pltpu.VMEM((CH, EMB_D), jnp.float32),
            pltpu.VMEM((CH, EMB_D), jnp.float32),
            pltpu.SemaphoreType.DMA,
            pltpu.SemaphoreType.DMA,
            pltpu.SemaphoreType.DMA,
            pltpu.SemaphoreType.DMA,
        ],
    )(tag_idx, tag_tab)


def _tc_body(x_ref, w_ref, b_ref, tag_ref, mask_ref, ptab_ref, out_ref):
    acc = jnp.dot(x_ref[...], w_ref[...], preferred_element_type=jnp.float32)
    out_ref[:, :PROJ_D] = acc + b_ref[...]
    out_ref[:, PROJ_D:PROJ_D + EMB_D] = tag_ref[...]
    pred = jnp.where(mask_ref[...] == 0, ptab_ref[0:1, :], ptab_ref[1:2, :])
    out_ref[:, PROJ_D + EMB_D:] = pred


def _tc_project_concat(x2d, W, b2d, tag_emb, mask_col, ptab):
    return pl.pallas_call(
        _tc_body,
        grid=(TOK // M_TILE,),
        in_specs=[
            pl.BlockSpec((M_TILE, IN_D), lambda i: (i, 0)),
            pl.BlockSpec((IN_D, PROJ_D), lambda i: (0, 0)),
            pl.BlockSpec((1, PROJ_D), lambda i: (0, 0)),
            pl.BlockSpec((M_TILE, EMB_D), lambda i: (i, 0)),
            pl.BlockSpec((M_TILE, 1), lambda i: (i, 0)),
            pl.BlockSpec((8, EMB_D), lambda i: (0, 0)),
        ],
        out_specs=pl.BlockSpec((M_TILE, OUT_D), lambda i: (i, 0)),
        out_shape=jax.ShapeDtypeStruct((TOK, OUT_D), jnp.float32),
    )(x2d, W, b2d, tag_emb, mask_col, ptab)


def kernel(input_layer, tag_ids, predicate_mask, tag_embeddings,
           predicate_embeddings, W, b):
    x2d = input_layer.reshape(TOK, IN_D)
    tag_idx = tag_ids.astype(jnp.int32).reshape(NW * NCH, CH)
    mask_col = predicate_mask.astype(jnp.int32).reshape(TOK, 1)
    ptab = jnp.zeros((8, EMB_D), jnp.float32).at[:2].set(predicate_embeddings)
    tag_emb = _sc_gather(tag_idx, tag_embeddings)
    out = _tc_project_concat(x2d, W, b.reshape(1, PROJ_D), tag_emb,
                             mask_col, ptab)
    return out.reshape(B, S, OUT_D)


# speedup vs baseline: 7.2839x; 1.0416x over previous
import functools

import jax
import jax.numpy as jnp
from jax import lax
from jax.experimental import pallas as pl
from jax.experimental.pallas import tpu as pltpu
from jax.experimental.pallas import tpu_sc as plsc

B, S = 4, 8192
TOK = B * S
IN_D = 768
PROJ_D = 512
EMB_D = 128
OUT_D = PROJ_D + 2 * EMB_D

NC, NS = 2, 16
NW = NC * NS
TPW = TOK // NW
CH = 128
NCH = TPW // CH
NBUF = 4

M_TILE = 2048


def _sc_gather_body(tag_idx_hbm, tag_tab_hbm, tag_out_hbm,
                    idx_t, r0, r1, r2, r3, s0, s1, s2, s3):
    wid = lax.axis_index("s") * NC + lax.axis_index("c")
    base = wid * NCH
    pltpu.sync_copy(tag_idx_hbm.at[pl.ds(base, NCH)], idx_t)
    rows = (r0, r1, r2, r3)
    sems = (s0, s1, s2, s3)

    def fire(j):
        return pltpu.async_copy(
            tag_tab_hbm.at[idx_t.at[j]], rows[j % NBUF], sems[j % NBUF])

    cps = {}
    for j in range(NBUF):
        cps[j] = fire(j)
    for j in range(NCH):
        cps.pop(j).wait()
        pltpu.sync_copy(rows[j % NBUF],
                        tag_out_hbm.at[pl.ds((base + j) * CH, CH)])
        if j + NBUF < NCH:
            cps[j + NBUF] = fire(j + NBUF)


def _sc_gather(tag_idx, tag_tab):
    mesh = plsc.VectorSubcoreMesh(core_axis_name="c", subcore_axis_name="s")
    return pl.kernel(
        _sc_gather_body,
        out_type=jax.ShapeDtypeStruct((TOK, EMB_D), jnp.float32),
        mesh=mesh,
        scratch_types=[
            pltpu.VMEM((NCH, CH), jnp.int32),
            pltpu.VMEM((CH, EMB_D), jnp.float32),
            pltpu.VMEM((CH, EMB_D), jnp.float32),
            pltpu.VMEM((CH, EMB_D), jnp.float32),
            pltpu.VMEM((CH, EMB_D), jnp.float32),
            pltpu.SemaphoreType.DMA,
            pltpu.SemaphoreType.DMA,
            pltpu.SemaphoreType.DMA,
            pltpu.SemaphoreType.DMA,
        ],
    )(tag_idx, tag_tab)


def _tc_body(x_ref, w_ref, b_ref, tag_ref, mask_ref, ptab_ref, out_ref):
    acc = jnp.dot(x_ref[...], w_ref[...], preferred_element_type=jnp.float32)
    out_ref[:, :PROJ_D] = acc + b_ref[...]
    out_ref[:, PROJ_D:PROJ_D + EMB_D] = tag_ref[...]
    pred = jnp.where(mask_ref[...] == 0, ptab_ref[0:1, :], ptab_ref[1:2, :])
    out_ref[:, PROJ_D + EMB_D:] = pred


def _tc_project_concat(x2d, W, b2d, tag_emb, mask_col, ptab):
    return pl.pallas_call(
        _tc_body,
        grid=(TOK // M_TILE,),
        in_specs=[
            pl.BlockSpec((M_TILE, IN_D), lambda i: (i, 0)),
            pl.BlockSpec((IN_D, PROJ_D), lambda i: (0, 0)),
            pl.BlockSpec((1, PROJ_D), lambda i: (0, 0)),
            pl.BlockSpec((M_TILE, EMB_D), lambda i: (i, 0)),
            pl.BlockSpec((M_TILE, 1), lambda i: (i, 0)),
            pl.BlockSpec((8, EMB_D), lambda i: (0, 0)),
        ],
        out_specs=pl.BlockSpec((M_TILE, OUT_D), lambda i: (i, 0)),
        out_shape=jax.ShapeDtypeStruct((TOK, OUT_D), jnp.float32),
    )(x2d, W, b2d, tag_emb, mask_col, ptab)


def kernel(input_layer, tag_ids, predicate_mask, tag_embeddings,
           predicate_embeddings, W, b):
    x2d = input_layer.reshape(TOK, IN_D)
    tag_idx = tag_ids.astype(jnp.int32).reshape(NW * NCH, CH)
    mask_col = predicate_mask.astype(jnp.int32).reshape(TOK, 1)
    ptab = jnp.zeros((8, EMB_D), jnp.float32).at[:2].set(predicate_embeddings)
    tag_emb = _sc_gather(tag_idx, tag_embeddings)
    out = _tc_project_concat(x2d, W, b.reshape(1, PROJ_D), tag_emb,
                             mask_col, ptab)
    return out.reshape(B, S, OUT_D)
